# baseline (speedup 1.0000x reference)
import functools

import jax
import jax.numpy as jnp
from jax import lax
from jax.experimental import pallas as pl
from jax.experimental.pallas import tpu as pltpu
from jax.experimental.pallas import tpu_sc as plsc

_ALPHA = 0.2
_NI = 16
_W = 2
_S = 512
_D = 256
_NH = 4
_GH = 16
_NEG = -9e15


def _leaky(x):
    return jnp.maximum(x, _ALPHA * x)


def _elu(x):
    return jnp.maximum(x, jnp.exp(jnp.minimum(x, 0.0)) - 1.0)


def _rollr(a, d):
    return pltpu.roll(a, (-d) % _S, 1) if d else a


def _dot(a, b):
    return jnp.dot(a, b, preferred_element_type=jnp.float32)


def _dgT(a, b):
    return jax.lax.dot_general(a, b, (((0,), (0,)), ((), ())),
                               preferred_element_type=jnp.float32)


def _dgTT(a, b):
    return jax.lax.dot_general(a, b, (((0,), (1,)), ((), ())),
                               preferred_element_type=jnp.float32)


def _dgNT(a, b):
    return jax.lax.dot_general(a, b, (((1,), (1,)), ((), ())),
                               preferred_element_type=jnp.float32)


def _band_rowmasks(valid_row, iota_row):
    masks = []
    for d in range(-_W, _W + 1):
        if d == 0:
            masks.append(None)
            continue
        inr = ((iota_row + d >= 0) & (iota_row + d < _S)).astype(jnp.float32)
        m = valid_row * _rollr(valid_row, d) * inr
        masks.append((1.0 - m) * _NEG)
    return masks


def _band_softmax_rows(src_t, dst_t, masks, extra_logits=None):
    lg = []
    for i, d in enumerate(range(-_W, _W + 1)):
        e = _leaky(src_t + _rollr(dst_t, d))
        if d != 0:
            e = e + masks[i]
        lg.append(e)
    m = lg[0]
    for e in lg[1:]:
        m = jnp.maximum(m, e)
    if extra_logits is not None:
        mex = jnp.concatenate(
            [jnp.max(ex, axis=0, keepdims=True) for ex in extra_logits],
            axis=0)
        m = jnp.maximum(m, mex)
    ps = [jnp.exp(e - m) for e in lg]
    den = ps[0]
    for p in ps[1:]:
        den = den + p
    pex = None
    if extra_logits is not None:
        pex = [jnp.exp(ex - m[h:h + 1, :]) for h, ex in enumerate(extra_logits)]
        den = den + jnp.concatenate(
            [jnp.sum(p, axis=0, keepdims=True) for p in pex], axis=0)
    rden = 1.0 / den
    return [p * rden for p in ps], pex, rden


_BPP = 8


def _sc_intent_mask(offs, batch):
    n = offs.shape[0]
    flat = batch * _NI
    mesh = plsc.VectorSubcoreMesh(core_axis_name="c", subcore_axis_name="s")

    @functools.partial(
        pl.kernel, mesh=mesh,
        out_type=jax.ShapeDtypeStruct((flat,), jnp.float32),
        scratch_types=[
            pltpu.VMEM((n,), jnp.int32),
            pltpu.VMEM((n,), jnp.float32),
            pltpu.VMEM((flat,), jnp.float32),
        ],
    )
    def k(offs_hbm, out_hbm, flat_v, ones_v, zeros_v):
        wid = lax.axis_index("s") * 2 + lax.axis_index("c")

        @pl.when(wid == 0)
        def _():
            pltpu.sync_copy(offs_hbm, flat_v)
            zeros_v[...] = jnp.zeros((flat,), jnp.float32)
            pltpu.sync_copy(zeros_v, out_hbm)
            ones_v[...] = jnp.ones((n,), jnp.float32)
            pltpu.sync_copy(ones_v, out_hbm.at[flat_v])

    return k(offs).reshape(batch, _NI)


def _gl_kernel(seq_ref, mask_ref,
               x_ref, iemb_ref,
               wss_ref, asss_ref, assd_ref, wos_ref, aos_ref,
               wsg_ref, asgs_ref, asgd_ref, wog_ref, aog_ref,
               w1_ref, b1_ref, w2_ref, b2_ref,
               out_ref):
    for j in range(_BPP):
        _gl_one(j, seq_ref, mask_ref, x_ref, iemb_ref,
                wss_ref, asss_ref, assd_ref, wos_ref, aos_ref,
                wsg_ref, asgs_ref, asgd_ref, wog_ref, aog_ref,
                w1_ref, b1_ref, w2_ref, b2_ref, out_ref)


def _gl_one(j, seq_ref, mask_ref,
            x_ref, iemb_ref,
            wss_ref, asss_ref, assd_ref, wos_ref, aos_ref,
            wsg_ref, asgs_ref, asgd_ref, wog_ref, aog_ref,
            w1_ref, b1_ref, w2_ref, b2_ref,
            out_ref):
    b = pl.program_id(0) * _BPP + j
    L = seq_ref[b]
    x_t = jnp.transpose(x_ref[j])

    iota_row = jax.lax.broadcasted_iota(jnp.int32, (1, _S), 1)
    valid_row = (iota_row < L).astype(jnp.float32)
    band_masks = _band_rowmasks(valid_row, iota_row)

    r4 = jax.lax.broadcasted_iota(jnp.int32, (_NH * _GH, _NH), 0) // _GH
    c4 = jax.lax.broadcasted_iota(jnp.int32, (_NH * _GH, _NH), 1)
    E4T = (r4 == c4).astype(jnp.float32)

    act_row = mask_ref[pl.ds(b, 1), :]
    act_col = jnp.transpose(act_row)

    h_all_t = _dot(wss_ref[...], x_t)
    src_t = _dot(asss_ref[...], h_all_t)
    dst_t = _dot(assd_ref[...], h_all_t)
    att, _, _ = _band_softmax_rows(src_t, dst_t, band_masks)
    hp_t = jnp.zeros((_NH * _GH, _S), jnp.float32)
    for i, d in enumerate(range(-_W, _W + 1)):
        hp_t = hp_t + _dot(E4T, att[i]) * _rollr(h_all_t, d)
    h1_t = _elu(hp_t)

    a_src_c = _dgNT(aos_ref[:, :_D], wos_ref[...])
    a_dst_c = _dgNT(aos_ref[:, _D:], wos_ref[...])
    src_t = _dot(a_src_c, h1_t)
    dst_t = _dot(a_dst_c, h1_t)
    att, _, _ = _band_softmax_rows(src_t, dst_t, band_masks)
    band = jnp.zeros((_NH * _GH, _S), jnp.float32)
    for i, d in enumerate(range(-_W, _W + 1)):
        band = band + att[i] * _rollr(h1_t, d)
    hp_o = _dgT(wos_ref[...], band)
    slot_out_t = _elu(hp_o) + x_t

    hg_I_t = _dgNT(wsg_ref[...], iemb_ref[...])
    hg_S_t = _dot(wsg_ref[...], slot_out_t)
    srcg_t = _dot(asgs_ref[...], hg_S_t)
    dstg_t = _dot(asgd_ref[...], hg_S_t)
    src_I = _dgTT(hg_I_t, asgs_ref[...])
    dst_I = _dgTT(hg_I_t, asgd_ref[...])
    dst_I_t = _dot(asgd_ref[...], hg_I_t)

    nm_IS = (1.0 - act_col * valid_row) * _NEG
    ints = []
    for k in range(_NH):
        ints.append(_leaky(srcg_t[k:k + 1, :] + dst_I[:, k:k + 1]) + nm_IS)
    att, pint, rden = _band_softmax_rows(srcg_t, dstg_t, band_masks, ints)
    hp_t = jnp.zeros((_NH * _GH, _S), jnp.float32)
    for i, d in enumerate(range(-_W, _W + 1)):
        hp_t = hp_t + _dot(E4T, att[i]) * _rollr(hg_S_t, d)
    hp_int = [_dot(hg_I_t[k * _GH:(k + 1) * _GH, :],
                   pint[k] * rden[k:k + 1, :]) for k in range(_NH)]
    hp_t = hp_t + jnp.concatenate(hp_int, axis=0)
    hg1_S_t = _elu(hp_t)

    eye = (jax.lax.broadcasted_iota(jnp.int32, (_NI, _NI), 0)
           == jax.lax.broadcasted_iota(jnp.int32, (_NI, _NI), 1))
    nm_II = jnp.where(
        jnp.logical_or((act_col * act_row) > 0, eye), 0.0, _NEG)
    hp_I = []
    for k in range(_NH):
        s_k = src_I[:, k:k + 1]
        lII = _leaky(s_k + dst_I_t[k:k + 1, :]) + nm_II
        lIS = _leaky(s_k + dstg_t[k:k + 1, :]) + nm_IS
        mI = jnp.maximum(jnp.max(lII, axis=1, keepdims=True),
                         jnp.max(lIS, axis=1, keepdims=True))
        pII = jnp.exp(lII - mI)
        pIS = jnp.exp(lIS - mI)
        rdenI = 1.0 / (jnp.sum(pII, axis=1, keepdims=True)
                       + jnp.sum(pIS, axis=1, keepdims=True))
        hp_I.append(_dgNT(hg_I_t[k * _GH:(k + 1) * _GH, :], pII * rdenI)
                    + _dgNT(hg_S_t[k * _GH:(k + 1) * _GH, :], pIS * rdenI))
    hg1_I_t = _elu(jnp.concatenate(hp_I, axis=0))

    g_src_c = _dgNT(aog_ref[:, :_D], wog_ref[...])
    g_dst_c = _dgNT(aog_ref[:, _D:], wog_ref[...])
    src_t = _dot(g_src_c, hg1_S_t)
    dst_t = _dot(g_dst_c, hg1_S_t)
    dst_go_I = _dgTT(hg1_I_t, g_dst_c)
    ei = _leaky(src_t + dst_go_I) + nm_IS
    att, pint, rden = _band_softmax_rows(src_t, dst_t, band_masks, [ei])
    band = jnp.zeros((_NH * _GH, _S), jnp.float32)
    for i, d in enumerate(range(-_W, _W + 1)):
        band = band + att[i] * _rollr(hg1_S_t, d)
    hgo_I_t = _dgT(wog_ref[...], hg1_I_t)
    hp_go = (_dgT(wog_ref[...], band)
             + _dot(hgo_I_t, pint[0] * rden))
    gout_t = _elu(hp_go) + slot_out_t

    b1_c = jnp.transpose(b1_ref[...])
    hf_t = _leaky(_dgT(w1_ref[...], gout_t) + b1_c)
    out_ref[j] = _dgT(hf_t, w2_ref[...]) + b2_ref[...]


def _full(shape):
    n = len(shape)
    return pl.BlockSpec(shape, lambda b, n=n: (0,) * n)


def kernel(hidden, seq_lens, intent_index, intent_embedding, Ws_slot, as_slot,
           Wo_slot, ao_slot, Ws_glob, as_glob, Wo_glob, ao_glob, W1, b1, W2, b2):
    B, S, D = hidden.shape
    OUT = W2.shape[1]

    NH, _, GH = Ws_slot.shape
    eye4 = jnp.eye(NH, dtype=jnp.float32)[:, :, None]

    def cat_t(Ws):
        return jnp.transpose(Ws, (0, 2, 1)).reshape(NH * GH, D)

    def bd_t(a_heads, half):
        sl = a_heads[None, :, half * GH:(half + 1) * GH]
        return (eye4 * sl).reshape(NH, NH * GH)

    offs = intent_index[:, 0] * _NI + intent_index[:, 1]
    mask = _sc_intent_mask(offs, B)
    args = (
        seq_lens, mask,
        hidden, intent_embedding,
        cat_t(Ws_slot), bd_t(as_slot, 0), bd_t(as_slot, 1),
        Wo_slot, ao_slot[None, :],
        cat_t(Ws_glob), bd_t(as_glob, 0), bd_t(as_glob, 1),
        Wo_glob, ao_glob[None, :],
        W1, b1[None, :], W2, b2[None, :],
    )
    in_specs = [pl.BlockSpec(memory_space=pltpu.SMEM)]
    in_specs.append(_full(mask.shape))
    in_specs.append(pl.BlockSpec((_BPP, S, D), lambda b: (b, 0, 0)))
    in_specs += [_full(a.shape) for a in args[3:]]

    return pl.pallas_call(
        _gl_kernel,
        grid=(B // _BPP,),
        in_specs=in_specs,
        out_specs=pl.BlockSpec((_BPP, S, OUT), lambda b: (b, 0, 0)),
        out_shape=jax.ShapeDtypeStruct((B, S, OUT), jnp.float32),
        compiler_params=pltpu.CompilerParams(
            dimension_semantics=("parallel",)),
    )(*args)


# speedup vs baseline: 1.3354x; 1.3354x over previous
import jax
import jax.numpy as jnp
from jax.experimental import pallas as pl
from jax.experimental.pallas import tpu as pltpu

_ALPHA = 0.2
_NI = 16
_W = 2
_S = 512
_D = 256
_NH = 4
_GH = 16
_NEG = -9e15


def _leaky(x):
    return jnp.maximum(x, _ALPHA * x)


def _elu(x):
    return jnp.maximum(x, jnp.exp(jnp.minimum(x, 0.0)) - 1.0)


def _rollr(a, d):
    return pltpu.roll(a, (-d) % _S, 1) if d else a


def _dot(a, b):
    return jnp.dot(a, b, preferred_element_type=jnp.float32)


def _dgT(a, b):
    return jax.lax.dot_general(a, b, (((0,), (0,)), ((), ())),
                               preferred_element_type=jnp.float32)


def _dgTT(a, b):
    return jax.lax.dot_general(a, b, (((0,), (1,)), ((), ())),
                               preferred_element_type=jnp.float32)


def _dgNT(a, b):
    return jax.lax.dot_general(a, b, (((1,), (1,)), ((), ())),
                               preferred_element_type=jnp.float32)


def _band_rowmasks(valid_row, iota_row):
    masks = []
    for d in range(-_W, _W + 1):
        if d == 0:
            masks.append(None)
            continue
        inr = ((iota_row + d >= 0) & (iota_row + d < _S)).astype(jnp.float32)
        m = valid_row * _rollr(valid_row, d) * inr
        masks.append((1.0 - m) * _NEG)
    return masks


def _band_softmax_rows(src_t, dst_t, masks, extra_logits=None):
    lg = []
    for i, d in enumerate(range(-_W, _W + 1)):
        e = _leaky(src_t + _rollr(dst_t, d))
        if d != 0:
            e = e + masks[i]
        lg.append(e)
    m = lg[0]
    for e in lg[1:]:
        m = jnp.maximum(m, e)
    if extra_logits is not None:
        mex = jnp.concatenate(
            [jnp.max(ex, axis=0, keepdims=True) for ex in extra_logits],
            axis=0)
        m = jnp.maximum(m, mex)
    ps = [jnp.exp(e - m) for e in lg]
    den = ps[0]
    for p in ps[1:]:
        den = den + p
    pex = None
    if extra_logits is not None:
        pex = [jnp.exp(ex - m[h:h + 1, :]) for h, ex in enumerate(extra_logits)]
        den = den + jnp.concatenate(
            [jnp.sum(p, axis=0, keepdims=True) for p in pex], axis=0)
    rden = 1.0 / den
    return [p * rden for p in ps], pex, rden


_BPP = 8


def _gl_kernel(seq_ref, idx_ref,
               x_ref, iemb_ref,
               wss_ref, asss_ref, assd_ref, wos_ref, aos_ref,
               wsg_ref, asgs_ref, asgd_ref, wog_ref, aog_ref,
               w1_ref, b1_ref, w2_ref, b2_ref,
               out_ref):
    for j in range(_BPP):
        _gl_one(j, seq_ref, idx_ref, x_ref, iemb_ref,
                wss_ref, asss_ref, assd_ref, wos_ref, aos_ref,
                wsg_ref, asgs_ref, asgd_ref, wog_ref, aog_ref,
                w1_ref, b1_ref, w2_ref, b2_ref, out_ref)


def _gl_one(j, seq_ref, idx_ref,
            x_ref, iemb_ref,
            wss_ref, asss_ref, assd_ref, wos_ref, aos_ref,
            wsg_ref, asgs_ref, asgd_ref, wog_ref, aog_ref,
            w1_ref, b1_ref, w2_ref, b2_ref,
            out_ref):
    b = pl.program_id(0) * _BPP + j
    L = seq_ref[b]
    x_t = jnp.transpose(x_ref[j])

    iota_row = jax.lax.broadcasted_iota(jnp.int32, (1, _S), 1)
    valid_row = (iota_row < L).astype(jnp.float32)
    band_masks = _band_rowmasks(valid_row, iota_row)

    r4 = jax.lax.broadcasted_iota(jnp.int32, (_NH * _GH, _NH), 0) // _GH
    c4 = jax.lax.broadcasted_iota(jnp.int32, (_NH * _GH, _NH), 1)
    E4T = (r4 == c4).astype(jnp.float32)

    i16r = jax.lax.broadcasted_iota(jnp.int32, (1, _NI), 1)
    hit = jnp.logical_and(idx_ref[:, 1:2] == i16r,
                          idx_ref[:, 0:1] == b)
    act_row = jnp.max(hit.astype(jnp.float32), axis=0, keepdims=True)
    act_col = jnp.transpose(act_row)

    h_all_t = _dot(wss_ref[...], x_t)
    src_t = _dot(asss_ref[...], h_all_t)
    dst_t = _dot(assd_ref[...], h_all_t)
    att, _, _ = _band_softmax_rows(src_t, dst_t, band_masks)
    hp_t = jnp.zeros((_NH * _GH, _S), jnp.float32)
    for i, d in enumerate(range(-_W, _W + 1)):
        hp_t = hp_t + _dot(E4T, att[i]) * _rollr(h_all_t, d)
    h1_t = _elu(hp_t)

    a_src_c = _dgNT(aos_ref[:, :_D], wos_ref[...])
    a_dst_c = _dgNT(aos_ref[:, _D:], wos_ref[...])
    src_t = _dot(a_src_c, h1_t)
    dst_t = _dot(a_dst_c, h1_t)
    att, _, _ = _band_softmax_rows(src_t, dst_t, band_masks)
    band = jnp.zeros((_NH * _GH, _S), jnp.float32)
    for i, d in enumerate(range(-_W, _W + 1)):
        band = band + att[i] * _rollr(h1_t, d)
    hp_o = _dgT(wos_ref[...], band)
    slot_out_t = _elu(hp_o) + x_t

    hg_I_t = _dgNT(wsg_ref[...], iemb_ref[...])
    hg_S_t = _dot(wsg_ref[...], slot_out_t)
    srcg_t = _dot(asgs_ref[...], hg_S_t)
    dstg_t = _dot(asgd_ref[...], hg_S_t)
    src_I = _dgTT(hg_I_t, asgs_ref[...])
    dst_I = _dgTT(hg_I_t, asgd_ref[...])
    dst_I_t = _dot(asgd_ref[...], hg_I_t)

    nm_IS = (1.0 - act_col * valid_row) * _NEG
    ints = []
    for k in range(_NH):
        ints.append(_leaky(srcg_t[k:k + 1, :] + dst_I[:, k:k + 1]) + nm_IS)
    att, pint, rden = _band_softmax_rows(srcg_t, dstg_t, band_masks, ints)
    hp_t = jnp.zeros((_NH * _GH, _S), jnp.float32)
    for i, d in enumerate(range(-_W, _W + 1)):
        hp_t = hp_t + _dot(E4T, att[i]) * _rollr(hg_S_t, d)
    hp_int = [_dot(hg_I_t[k * _GH:(k + 1) * _GH, :],
                   pint[k] * rden[k:k + 1, :]) for k in range(_NH)]
    hp_t = hp_t + jnp.concatenate(hp_int, axis=0)
    hg1_S_t = _elu(hp_t)

    eye = (jax.lax.broadcasted_iota(jnp.int32, (_NI, _NI), 0)
           == jax.lax.broadcasted_iota(jnp.int32, (_NI, _NI), 1))
    nm_II = jnp.where(
        jnp.logical_or((act_col * act_row) > 0, eye), 0.0, _NEG)
    hp_I = []
    for k in range(_NH):
        s_k = src_I[:, k:k + 1]
        lII = _leaky(s_k + dst_I_t[k:k + 1, :]) + nm_II
        lIS = _leaky(s_k + dstg_t[k:k + 1, :]) + nm_IS
        mI = jnp.maximum(jnp.max(lII, axis=1, keepdims=True),
                         jnp.max(lIS, axis=1, keepdims=True))
        pII = jnp.exp(lII - mI)
        pIS = jnp.exp(lIS - mI)
        rdenI = 1.0 / (jnp.sum(pII, axis=1, keepdims=True)
                       + jnp.sum(pIS, axis=1, keepdims=True))
        hp_I.append(_dgNT(hg_I_t[k * _GH:(k + 1) * _GH, :], pII * rdenI)
                    + _dgNT(hg_S_t[k * _GH:(k + 1) * _GH, :], pIS * rdenI))
    hg1_I_t = _elu(jnp.concatenate(hp_I, axis=0))

    g_src_c = _dgNT(aog_ref[:, :_D], wog_ref[...])
    g_dst_c = _dgNT(aog_ref[:, _D:], wog_ref[...])
    src_t = _dot(g_src_c, hg1_S_t)
    dst_t = _dot(g_dst_c, hg1_S_t)
    dst_go_I = _dgTT(hg1_I_t, g_dst_c)
    ei = _leaky(src_t + dst_go_I) + nm_IS
    att, pint, rden = _band_softmax_rows(src_t, dst_t, band_masks, [ei])
    band = jnp.zeros((_NH * _GH, _S), jnp.float32)
    for i, d in enumerate(range(-_W, _W + 1)):
        band = band + att[i] * _rollr(hg1_S_t, d)
    hgo_I_t = _dgT(wog_ref[...], hg1_I_t)
    hp_go = (_dgT(wog_ref[...], band)
             + _dot(hgo_I_t, pint[0] * rden))
    gout_t = _elu(hp_go) + slot_out_t

    b1_c = jnp.transpose(b1_ref[...])
    hf_t = _leaky(_dgT(w1_ref[...], gout_t) + b1_c)
    out_ref[j] = _dgT(hf_t, w2_ref[...]) + b2_ref[...]


def _full(shape):
    n = len(shape)
    return pl.BlockSpec(shape, lambda b, n=n: (0,) * n)


def kernel(hidden, seq_lens, intent_index, intent_embedding, Ws_slot, as_slot,
           Wo_slot, ao_slot, Ws_glob, as_glob, Wo_glob, ao_glob, W1, b1, W2, b2):
    B, S, D = hidden.shape
    OUT = W2.shape[1]

    NH, _, GH = Ws_slot.shape
    eye4 = jnp.eye(NH, dtype=jnp.float32)[:, :, None]

    def cat_t(Ws):
        return jnp.transpose(Ws, (0, 2, 1)).reshape(NH * GH, D)

    def bd_t(a_heads, half):
        sl = a_heads[None, :, half * GH:(half + 1) * GH]
        return (eye4 * sl).reshape(NH, NH * GH)

    args = (
        seq_lens, intent_index,
        hidden, intent_embedding,
        cat_t(Ws_slot), bd_t(as_slot, 0), bd_t(as_slot, 1),
        Wo_slot, ao_slot[None, :],
        cat_t(Ws_glob), bd_t(as_glob, 0), bd_t(as_glob, 1),
        Wo_glob, ao_glob[None, :],
        W1, b1[None, :], W2, b2[None, :],
    )
    in_specs = [pl.BlockSpec(memory_space=pltpu.SMEM)]
    in_specs.append(_full(intent_index.shape))
    in_specs.append(pl.BlockSpec((_BPP, S, D), lambda b: (b, 0, 0)))
    in_specs += [_full(a.shape) for a in args[3:]]

    return pl.pallas_call(
        _gl_kernel,
        grid=(B // _BPP,),
        in_specs=in_specs,
        out_specs=pl.BlockSpec((_BPP, S, OUT), lambda b: (b, 0, 0)),
        out_shape=jax.ShapeDtypeStruct((B, S, OUT), jnp.float32),
        compiler_params=pltpu.CompilerParams(
            dimension_semantics=("parallel",)),
    )(*args)


# speedup vs baseline: 1.4167x; 1.0609x over previous
import jax
import jax.numpy as jnp
from jax.experimental import pallas as pl
from jax.experimental.pallas import tpu as pltpu

_ALPHA = 0.2
_NI = 16
_W = 2
_S = 512
_D = 256
_NH = 4
_GH = 16
_NEG = -9e15


def _leaky(x):
    return jnp.maximum(x, _ALPHA * x)


def _elu(x):
    return jnp.maximum(x, jnp.exp(jnp.minimum(x, 0.0)) - 1.0)


def _rollr(a, d):
    return pltpu.roll(a, (-d) % _S, 1) if d else a


def _dot(a, b):
    return jnp.dot(a, b, preferred_element_type=jnp.float32)


def _dgT(a, b):
    return jax.lax.dot_general(a, b, (((0,), (0,)), ((), ())),
                               preferred_element_type=jnp.float32)


def _dgTT(a, b):
    return jax.lax.dot_general(a, b, (((0,), (1,)), ((), ())),
                               preferred_element_type=jnp.float32)


def _dgNT(a, b):
    return jax.lax.dot_general(a, b, (((1,), (1,)), ((), ())),
                               preferred_element_type=jnp.float32)


def _band_rowmasks(valid_row, iota_row):
    masks = []
    for d in range(-_W, _W + 1):
        if d == 0:
            masks.append(None)
            continue
        inr = ((iota_row + d >= 0) & (iota_row + d < _S)).astype(jnp.float32)
        m = valid_row * _rollr(valid_row, d) * inr
        masks.append((1.0 - m) * _NEG)
    return masks


def _band_softmax_rows(src_t, dst_t, masks, extra_logits=None):
    lg = []
    for i, d in enumerate(range(-_W, _W + 1)):
        e = _leaky(src_t + _rollr(dst_t, d))
        if d != 0:
            e = e + masks[i]
        lg.append(e)
    m = lg[0]
    for e in lg[1:]:
        m = jnp.maximum(m, e)
    if extra_logits is not None:
        mex = jnp.concatenate(
            [jnp.max(ex, axis=0, keepdims=True) for ex in extra_logits],
            axis=0)
        m = jnp.maximum(m, mex)
    ps = [jnp.exp(e - m) for e in lg]
    den = ps[0]
    for p in ps[1:]:
        den = den + p
    pex = None
    if extra_logits is not None:
        pex = [jnp.exp(ex - m[h:h + 1, :]) for h, ex in enumerate(extra_logits)]
        den = den + jnp.concatenate(
            [jnp.sum(p, axis=0, keepdims=True) for p in pex], axis=0)
    rden = 1.0 / den
    return [p * rden for p in ps], pex, rden


def _band_apply_heads(att, h_all_t):
    rows = []
    for k in range(_NH):
        h_k = h_all_t[k * _GH:(k + 1) * _GH, :]
        acc = att[_W][k:k + 1, :] * h_k
        for i, d in enumerate(range(-_W, _W + 1)):
            if d != 0:
                acc = acc + att[i][k:k + 1, :] * _rollr(h_k, d)
        rows.append(acc)
    return jnp.concatenate(rows, axis=0)


_BPP = 8


def _gl_kernel(seq_ref, idx_ref,
               x_ref, iemb_ref,
               wss_ref, asss_ref, assd_ref, wos_ref, aos_ref,
               wsg_ref, asgs_ref, asgd_ref, wog_ref, aog_ref,
               w1_ref, b1_ref, w2_ref, b2_ref,
               out_ref):
    for j in range(_BPP):
        _gl_one(j, seq_ref, idx_ref, x_ref, iemb_ref,
                wss_ref, asss_ref, assd_ref, wos_ref, aos_ref,
                wsg_ref, asgs_ref, asgd_ref, wog_ref, aog_ref,
                w1_ref, b1_ref, w2_ref, b2_ref, out_ref)


def _gl_one(j, seq_ref, idx_ref,
            x_ref, iemb_ref,
            wss_ref, asss_ref, assd_ref, wos_ref, aos_ref,
            wsg_ref, asgs_ref, asgd_ref, wog_ref, aog_ref,
            w1_ref, b1_ref, w2_ref, b2_ref,
            out_ref):
    b = pl.program_id(0) * _BPP + j
    L = seq_ref[b]
    x_t = jnp.transpose(x_ref[j])

    iota_row = jax.lax.broadcasted_iota(jnp.int32, (1, _S), 1)
    valid_row = (iota_row < L).astype(jnp.float32)
    band_masks = _band_rowmasks(valid_row, iota_row)

    i16r = jax.lax.broadcasted_iota(jnp.int32, (1, _NI), 1)
    hit = jnp.logical_and(idx_ref[:, 1:2] == i16r,
                          idx_ref[:, 0:1] == b)
    act_row = jnp.max(hit.astype(jnp.float32), axis=0, keepdims=True)
    act_col = jnp.transpose(act_row)

    h_all_t = _dot(wss_ref[...], x_t)
    src_t = _dot(asss_ref[...], h_all_t)
    dst_t = _dot(assd_ref[...], h_all_t)
    att, _, _ = _band_softmax_rows(src_t, dst_t, band_masks)
    h1_t = _elu(_band_apply_heads(att, h_all_t))

    a_src_c = _dgNT(aos_ref[:, :_D], wos_ref[...])
    a_dst_c = _dgNT(aos_ref[:, _D:], wos_ref[...])
    src_t = _dot(a_src_c, h1_t)
    dst_t = _dot(a_dst_c, h1_t)
    att, _, _ = _band_softmax_rows(src_t, dst_t, band_masks)
    band = jnp.zeros((_NH * _GH, _S), jnp.float32)
    for i, d in enumerate(range(-_W, _W + 1)):
        band = band + att[i] * _rollr(h1_t, d)
    hp_o = _dgT(wos_ref[...], band)
    slot_out_t = _elu(hp_o) + x_t

    hg_I_t = _dgNT(wsg_ref[...], iemb_ref[...])
    hg_S_t = _dot(wsg_ref[...], slot_out_t)
    srcg_t = _dot(asgs_ref[...], hg_S_t)
    dstg_t = _dot(asgd_ref[...], hg_S_t)
    src_I = _dgTT(hg_I_t, asgs_ref[...])
    dst_I = _dgTT(hg_I_t, asgd_ref[...])
    dst_I_t = _dot(asgd_ref[...], hg_I_t)

    nm_IS = (1.0 - act_col * valid_row) * _NEG
    ints = []
    for k in range(_NH):
        ints.append(_leaky(srcg_t[k:k + 1, :] + dst_I[:, k:k + 1]) + nm_IS)
    att, pint, rden = _band_softmax_rows(srcg_t, dstg_t, band_masks, ints)
    hp_t = _band_apply_heads(att, hg_S_t)
    hp_int = [_dot(hg_I_t[k * _GH:(k + 1) * _GH, :],
                   pint[k] * rden[k:k + 1, :]) for k in range(_NH)]
    hp_t = hp_t + jnp.concatenate(hp_int, axis=0)
    hg1_S_t = _elu(hp_t)

    eye = (jax.lax.broadcasted_iota(jnp.int32, (_NI, _NI), 0)
           == jax.lax.broadcasted_iota(jnp.int32, (_NI, _NI), 1))
    nm_II = jnp.where(
        jnp.logical_or((act_col * act_row) > 0, eye), 0.0, _NEG)
    hp_I = []
    for k in range(_NH):
        s_k = src_I[:, k:k + 1]
        lII = _leaky(s_k + dst_I_t[k:k + 1, :]) + nm_II
        lIS = _leaky(s_k + dstg_t[k:k + 1, :]) + nm_IS
        mI = jnp.maximum(jnp.max(lII, axis=1, keepdims=True),
                         jnp.max(lIS, axis=1, keepdims=True))
        pII = jnp.exp(lII - mI)
        pIS = jnp.exp(lIS - mI)
        rdenI = 1.0 / (jnp.sum(pII, axis=1, keepdims=True)
                       + jnp.sum(pIS, axis=1, keepdims=True))
        hp_I.append(_dgNT(hg_I_t[k * _GH:(k + 1) * _GH, :], pII * rdenI)
                    + _dgNT(hg_S_t[k * _GH:(k + 1) * _GH, :], pIS * rdenI))
    hg1_I_t = _elu(jnp.concatenate(hp_I, axis=0))

    g_src_c = _dgNT(aog_ref[:, :_D], wog_ref[...])
    g_dst_c = _dgNT(aog_ref[:, _D:], wog_ref[...])
    src_t = _dot(g_src_c, hg1_S_t)
    dst_t = _dot(g_dst_c, hg1_S_t)
    dst_go_I = _dgTT(hg1_I_t, g_dst_c)
    ei = _leaky(src_t + dst_go_I) + nm_IS
    att, pint, rden = _band_softmax_rows(src_t, dst_t, band_masks, [ei])
    band = jnp.zeros((_NH * _GH, _S), jnp.float32)
    for i, d in enumerate(range(-_W, _W + 1)):
        band = band + att[i] * _rollr(hg1_S_t, d)
    hgo_I_t = _dgT(wog_ref[...], hg1_I_t)
    hp_go = (_dgT(wog_ref[...], band)
             + _dot(hgo_I_t, pint[0] * rden))
    gout_t = _elu(hp_go) + slot_out_t

    b1_c = jnp.transpose(b1_ref[...])
    hf_t = _leaky(_dgT(w1_ref[...], gout_t) + b1_c)
    out_ref[j] = _dgT(hf_t, w2_ref[...]) + b2_ref[...]


def _full(shape):
    n = len(shape)
    return pl.BlockSpec(shape, lambda b, n=n: (0,) * n)


def kernel(hidden, seq_lens, intent_index, intent_embedding, Ws_slot, as_slot,
           Wo_slot, ao_slot, Ws_glob, as_glob, Wo_glob, ao_glob, W1, b1, W2, b2):
    B, S, D = hidden.shape
    OUT = W2.shape[1]

    NH, _, GH = Ws_slot.shape
    eye4 = jnp.eye(NH, dtype=jnp.float32)[:, :, None]

    def cat_t(Ws):
        return jnp.transpose(Ws, (0, 2, 1)).reshape(NH * GH, D)

    def bd_t(a_heads, half):
        sl = a_heads[None, :, half * GH:(half + 1) * GH]
        return (eye4 * sl).reshape(NH, NH * GH)

    args = (
        seq_lens, intent_index,
        hidden, intent_embedding,
        cat_t(Ws_slot), bd_t(as_slot, 0), bd_t(as_slot, 1),
        Wo_slot, ao_slot[None, :],
        cat_t(Ws_glob), bd_t(as_glob, 0), bd_t(as_glob, 1),
        Wo_glob, ao_glob[None, :],
        W1, b1[None, :], W2, b2[None, :],
    )
    in_specs = [pl.BlockSpec(memory_space=pltpu.SMEM)]
    in_specs.append(_full(intent_index.shape))
    in_specs.append(pl.BlockSpec((_BPP, S, D), lambda b: (b, 0, 0)))
    in_specs += [_full(a.shape) for a in args[3:]]

    return pl.pallas_call(
        _gl_kernel,
        grid=(B // _BPP,),
        in_specs=in_specs,
        out_specs=pl.BlockSpec((_BPP, S, OUT), lambda b: (b, 0, 0)),
        out_shape=jax.ShapeDtypeStruct((B, S, OUT), jnp.float32),
        compiler_params=pltpu.CompilerParams(
            dimension_semantics=("parallel",)),
    )(*args)
